# baseline (speedup 1.0000x reference)
import functools

import jax
import jax.numpy as jnp
from jax import lax
from jax.experimental import pallas as pl
from jax.experimental.pallas import tpu as pltpu
from jax.experimental.pallas import tpu_sc as plsc

N = 131072
C = 150
CP = 256
RB = 1024
ROWS_2D = N // 128

NC = 2
NS = 16
NW = NC * NS
RPW = N // NW
G = 128
CHUNKS = RPW // G


def _entropy_body(params_ref, x_ref, g_ref, pre_ref, cond_ref):
    x = x_ref[...]
    thr = params_ref[0, 0]
    invt = params_ref[0, 1]
    m = jnp.max(x, axis=1, keepdims=True)
    e = jnp.exp(x - m)
    s = jnp.sum(e, axis=1, keepdims=True)
    t = jnp.sum(e * (x - m), axis=1, keepdims=True)
    ent = jnp.log(s) - t / s
    cond = ent < thr
    row = jnp.where(cond, x * invt, jnp.float32(1.0))
    gbits = lax.bitcast_convert_type(g_ref[...], jnp.float32)
    pad = jnp.zeros((RB, CP - C - 1), jnp.float32)
    pre_ref[...] = jnp.concatenate([row, gbits, pad], axis=1)
    cond_ref[...] = cond.astype(jnp.int32)


def _dest_body(cond_ref, dest_ref):
    c = cond_ref[...].astype(jnp.float32)
    r, l = ROWS_2D, 128
    u = (lax.broadcasted_iota(jnp.int32, (l, l), 0)
         <= lax.broadcasted_iota(jnp.int32, (l, l), 1)).astype(jnp.float32)
    cs = lax.dot_general(c, u, (((1,), (0,)), ((), ())))
    rs = jnp.sum(c, axis=1, keepdims=True)
    lo = (lax.broadcasted_iota(jnp.int32, (r, r), 0)
          > lax.broadcasted_iota(jnp.int32, (r, r), 1)).astype(jnp.float32)
    ro = lax.dot_general(lo, rs, (((1,), (0,)), ((), ())))
    cc = cs + ro
    k = jnp.max(cc)
    v = (lax.broadcasted_iota(jnp.int32, (r, l), 0) * l
         + lax.broadcasted_iota(jnp.int32, (r, l), 1)).astype(jnp.float32)
    dest = jnp.where(c > 0.5, cc - 1.0, k + v - cc)
    dest_ref[...] = dest.astype(jnp.int32)


def _sc_invperm_body(dest_hbm, src_hbm, idx_v, val_v, sem):
    wid = lax.axis_index("s") * NC + lax.axis_index("c")
    base0 = wid * RPW

    def chunk(i, carry):
        base = base0 + i * G
        pltpu.sync_copy(dest_hbm.at[pl.ds(base, G)], idx_v)
        for j in range(G // 16):
            val_v[pl.ds(j * 16, 16)] = (
                lax.iota(jnp.int32, 16) + base + j * 16)
        pltpu.async_copy(val_v, src_hbm.at[idx_v], sem).wait()
        return carry

    lax.fori_loop(0, CHUNKS, chunk, 0)


def _sc_gather_body(pre_hbm, src_hbm, out_hbm, outgt_hbm,
                    rows_v, idx_v, gt_v, sem):
    wid = lax.axis_index("s") * NC + lax.axis_index("c")
    base0 = wid * RPW

    def chunk(i, carry):
        base = base0 + i * G
        pltpu.sync_copy(src_hbm.at[pl.ds(base, G)], idx_v)
        pltpu.async_copy(pre_hbm.at[idx_v], rows_v, sem).wait()
        for j in range(G // 16):
            r16 = lax.iota(jnp.int32, 16) + j * 16
            l16 = jnp.full((16,), C, jnp.int32)
            bits = plsc.load_gather(rows_v, [r16, l16])
            gt_v[pl.ds(j * 16, 16)] = lax.bitcast_convert_type(bits, jnp.int32)
        pltpu.sync_copy(rows_v, out_hbm.at[pl.ds(base, G)])
        pltpu.sync_copy(gt_v, outgt_hbm.at[pl.ds(base, G)])
        return carry

    lax.fori_loop(0, CHUNKS, chunk, 0)


@functools.cache
def _sc_kernels():
    mesh = plsc.VectorSubcoreMesh(core_axis_name="c", subcore_axis_name="s")
    cp = pltpu.CompilerParams(needs_layout_passes=False)
    invperm = pl.kernel(
        _sc_invperm_body,
        out_type=jax.ShapeDtypeStruct((N,), jnp.int32),
        mesh=mesh,
        compiler_params=cp,
        scratch_types=[
            pltpu.VMEM((G,), jnp.int32),
            pltpu.VMEM((G,), jnp.int32),
            pltpu.SemaphoreType.DMA,
        ],
    )
    gather = pl.kernel(
        _sc_gather_body,
        out_type=[
            jax.ShapeDtypeStruct((N, CP), jnp.float32),
            jax.ShapeDtypeStruct((N,), jnp.int32),
        ],
        mesh=mesh,
        compiler_params=cp,
        scratch_types=[
            pltpu.VMEM((G, CP), jnp.float32),
            pltpu.VMEM((G,), jnp.int32),
            pltpu.VMEM((G,), jnp.int32),
            pltpu.SemaphoreType.DMA,
        ],
    )
    return invperm, gather


def kernel(logits, gt, threshold, temperature_single):
    x2 = jnp.transpose(logits, (0, 2, 3, 1)).reshape(N, C)
    y2 = gt.reshape(N, 1)
    thr = jnp.asarray(threshold, jnp.float32)
    invt = jnp.float32(1.0) / temperature_single[0].astype(jnp.float32)
    params = jnp.stack([thr, invt]).reshape(1, 2)

    pre, cond = pl.pallas_call(
        _entropy_body,
        grid=(N // RB,),
        in_specs=[
            pl.BlockSpec(memory_space=pltpu.SMEM),
            pl.BlockSpec((RB, C), lambda i: (i, 0)),
            pl.BlockSpec((RB, 1), lambda i: (i, 0)),
        ],
        out_specs=[
            pl.BlockSpec((RB, CP), lambda i: (i, 0)),
            pl.BlockSpec((RB, 1), lambda i: (i, 0)),
        ],
        out_shape=[
            jax.ShapeDtypeStruct((N, CP), jnp.float32),
            jax.ShapeDtypeStruct((N, 1), jnp.int32),
        ],
    )(params, x2, y2)

    dest2d = pl.pallas_call(
        _dest_body,
        in_specs=[pl.BlockSpec((ROWS_2D, 128), lambda: (0, 0))],
        out_specs=pl.BlockSpec((ROWS_2D, 128), lambda: (0, 0)),
        out_shape=jax.ShapeDtypeStruct((ROWS_2D, 128), jnp.int32),
    )(cond.reshape(ROWS_2D, 128))

    invperm, gather = _sc_kernels()
    src = invperm(dest2d.reshape(N))
    out_pad, cal_gt = gather(pre, src)
    cal_logits = out_pad[:, :C]
    return (cal_logits, cal_gt)


# speedup vs baseline: 1.3883x; 1.3883x over previous
import functools

import jax
import jax.numpy as jnp
from jax import lax
from jax.experimental import pallas as pl
from jax.experimental.pallas import tpu as pltpu
from jax.experimental.pallas import tpu_sc as plsc

N = 131072
C = 150
CP = 256
RB = 1024
ROWS_2D = N // 128

NC = 2
NS = 16
NW = NC * NS
RPW = N // NW
G = 128
CHUNKS = RPW // G


def _entropy_body(params_ref, x_ref, g_ref, pre_ref, cond_ref):
    x = jnp.concatenate(
        [jnp.transpose(x_ref[0, :, h, :]) for h in range(8)], axis=0)
    thr = params_ref[0, 0]
    invt = params_ref[0, 1]
    m = jnp.max(x, axis=1, keepdims=True)
    e = jnp.exp(x - m)
    s = jnp.sum(e, axis=1, keepdims=True)
    t = jnp.sum(e * (x - m), axis=1, keepdims=True)
    ent = jnp.log(s) - t / s
    cond = ent < thr
    row = jnp.where(cond, x * invt, jnp.float32(1.0))
    gbits = lax.bitcast_convert_type(g_ref[...], jnp.float32)
    pad = jnp.zeros((RB, CP - C - 1), jnp.float32)
    pre_ref[...] = jnp.concatenate([row, gbits, pad], axis=1)
    cond_ref[...] = cond.astype(jnp.int32)


def _dest_body(cond_ref, dest_ref):
    c = cond_ref[...].astype(jnp.float32)
    r, l = ROWS_2D, 128
    u = (lax.broadcasted_iota(jnp.int32, (l, l), 0)
         <= lax.broadcasted_iota(jnp.int32, (l, l), 1)).astype(jnp.float32)
    cs = lax.dot_general(c, u, (((1,), (0,)), ((), ())))
    rs = jnp.sum(c, axis=1, keepdims=True)
    lo = (lax.broadcasted_iota(jnp.int32, (r, r), 0)
          > lax.broadcasted_iota(jnp.int32, (r, r), 1)).astype(jnp.float32)
    ro = lax.dot_general(lo, rs, (((1,), (0,)), ((), ())))
    cc = cs + ro
    k = jnp.max(cc)
    v = (lax.broadcasted_iota(jnp.int32, (r, l), 0) * l
         + lax.broadcasted_iota(jnp.int32, (r, l), 1)).astype(jnp.float32)
    dest = jnp.where(c > 0.5, cc - 1.0, k + v - cc)
    dest_ref[...] = dest.astype(jnp.int32)


def _sc_scatter_body(pre_hbm, dest_hbm, out_hbm, rows_v, idx_v, sem):
    wid = lax.axis_index("s") * NC + lax.axis_index("c")
    base0 = wid * RPW

    def chunk(i, carry):
        base = base0 + i * G
        pltpu.sync_copy(dest_hbm.at[pl.ds(base, G)], idx_v)
        pltpu.sync_copy(pre_hbm.at[pl.ds(base, G)], rows_v)
        pltpu.async_copy(rows_v, out_hbm.at[idx_v], sem).wait()
        return carry

    lax.fori_loop(0, CHUNKS, chunk, 0)


@functools.cache
def _sc_scatter():
    return pl.kernel(
        _sc_scatter_body,
        out_type=jax.ShapeDtypeStruct((N, CP), jnp.float32),
        mesh=plsc.VectorSubcoreMesh(core_axis_name="c", subcore_axis_name="s"),
        scratch_types=[
            pltpu.VMEM((G, CP), jnp.float32),
            pltpu.VMEM((G,), jnp.int32),
            pltpu.SemaphoreType.DMA,
        ],
    )


def _extract_body(pad_ref, out_ref, gt_ref):
    out_ref[...] = pad_ref[:, :C]
    gt_ref[...] = lax.bitcast_convert_type(pad_ref[:, C:C + 1], jnp.int32)


def kernel(logits, gt, threshold, temperature_single):
    y2 = gt.reshape(N, 1)
    thr = jnp.asarray(threshold, jnp.float32)
    invt = jnp.float32(1.0) / temperature_single[0].astype(jnp.float32)
    params = jnp.stack([thr, invt]).reshape(1, 2)

    pre, cond = pl.pallas_call(
        _entropy_body,
        grid=(8, 16),
        in_specs=[
            pl.BlockSpec(memory_space=pltpu.SMEM),
            pl.BlockSpec((1, C, 8, 128), lambda b, i: (b, 0, i, 0)),
            pl.BlockSpec((RB, 1), lambda b, i: (b * 16 + i, 0)),
        ],
        out_specs=[
            pl.BlockSpec((RB, CP), lambda b, i: (b * 16 + i, 0)),
            pl.BlockSpec((RB, 1), lambda b, i: (b * 16 + i, 0)),
        ],
        out_shape=[
            jax.ShapeDtypeStruct((N, CP), jnp.float32),
            jax.ShapeDtypeStruct((N, 1), jnp.int32),
        ],
    )(params, logits, y2)

    dest2d = pl.pallas_call(
        _dest_body,
        in_specs=[pl.BlockSpec((ROWS_2D, 128), lambda: (0, 0))],
        out_specs=pl.BlockSpec((ROWS_2D, 128), lambda: (0, 0)),
        out_shape=jax.ShapeDtypeStruct((ROWS_2D, 128), jnp.int32),
    )(cond.reshape(ROWS_2D, 128))

    out_pad = _sc_scatter()(pre, dest2d.reshape(N))

    cal_logits, cal_gt2d = pl.pallas_call(
        _extract_body,
        grid=(N // RB,),
        in_specs=[pl.BlockSpec((RB, CP), lambda i: (i, 0))],
        out_specs=[
            pl.BlockSpec((RB, C), lambda i: (i, 0)),
            pl.BlockSpec((RB, 1), lambda i: (i, 0)),
        ],
        out_shape=[
            jax.ShapeDtypeStruct((N, C), jnp.float32),
            jax.ShapeDtypeStruct((N, 1), jnp.int32),
        ],
    )(out_pad)
    return (cal_logits, cal_gt2d.reshape(N))


# speedup vs baseline: 1.8582x; 1.3385x over previous
import functools

import jax
import jax.numpy as jnp
from jax import lax
from jax.experimental import pallas as pl
from jax.experimental.pallas import tpu as pltpu
from jax.experimental.pallas import tpu_sc as plsc

N = 131072
C = 150
CP = 256
RB = 1024
ROWS_2D = N // 128

NC = 2
NS = 16
NW = NC * NS
RPW = N // NW
G = 128
CHUNKS = RPW // G


def _entropy_body(params_ref, x_ref, g_ref, pre_ref, cond_ref):
    x = jnp.concatenate(
        [jnp.transpose(x_ref[0, :, h, :]) for h in range(8)], axis=0)
    thr = params_ref[0, 0]
    invt = params_ref[0, 1]
    m = jnp.max(x, axis=1, keepdims=True)
    e = jnp.exp(x - m)
    s = jnp.sum(e, axis=1, keepdims=True)
    t = jnp.sum(e * (x - m), axis=1, keepdims=True)
    ent = jnp.log(s) - t / s
    cond = ent < thr
    row = jnp.where(cond, x * invt, jnp.float32(1.0))
    gbits = lax.bitcast_convert_type(g_ref[...], jnp.float32)
    pad = jnp.zeros((RB, CP - C - 1), jnp.float32)
    pre_ref[...] = jnp.concatenate([row, gbits, pad], axis=1)
    cond_ref[...] = cond.astype(jnp.int32)


def _dest_body(cond_ref, dest_ref):
    c = cond_ref[...].astype(jnp.float32)
    r, l = ROWS_2D, 128
    u = (lax.broadcasted_iota(jnp.int32, (l, l), 0)
         <= lax.broadcasted_iota(jnp.int32, (l, l), 1)).astype(jnp.float32)
    cs = lax.dot_general(c, u, (((1,), (0,)), ((), ())))
    rs = jnp.sum(c, axis=1, keepdims=True)
    lo = (lax.broadcasted_iota(jnp.int32, (r, r), 0)
          > lax.broadcasted_iota(jnp.int32, (r, r), 1)).astype(jnp.float32)
    ro = lax.dot_general(lo, rs, (((1,), (0,)), ((), ())))
    cc = cs + ro
    k = jnp.max(cc)
    v = (lax.broadcasted_iota(jnp.int32, (r, l), 0) * l
         + lax.broadcasted_iota(jnp.int32, (r, l), 1)).astype(jnp.float32)
    dest = jnp.where(c > 0.5, cc - 1.0, k + v - cc)
    dest_ref[...] = dest.astype(jnp.int32)


def _sc_scatter_body(pre_hbm, dest_hbm, out_hbm, rows_v, idx_v, sem):
    wid = lax.axis_index("s") * NC + lax.axis_index("c")
    base0 = wid * RPW

    def chunk(i, carry):
        base = base0 + i * G
        pltpu.sync_copy(dest_hbm.at[pl.ds(base, G)], idx_v)
        pltpu.sync_copy(pre_hbm.at[pl.ds(base, G)], rows_v)
        pltpu.async_copy(rows_v, out_hbm.at[idx_v], sem).wait()
        return carry

    lax.fori_loop(0, CHUNKS, chunk, 0)


@functools.cache
def _sc_scatter():
    return pl.kernel(
        _sc_scatter_body,
        out_type=jax.ShapeDtypeStruct((N, CP), jnp.float32),
        mesh=plsc.VectorSubcoreMesh(core_axis_name="c", subcore_axis_name="s"),
        scratch_types=[
            pltpu.VMEM((G, CP), jnp.float32),
            pltpu.VMEM((G,), jnp.int32),
            pltpu.SemaphoreType.DMA,
        ],
    )


def _extract_body(pad_ref, out_ref, gt_ref):
    out_ref[...] = pad_ref[:, :C]
    gt_ref[...] = lax.bitcast_convert_type(pad_ref[:, C:C + 1], jnp.int32)


def kernel(logits, gt, threshold, temperature_single):
    y2 = gt.reshape(N, 1)
    thr = jnp.asarray(threshold, jnp.float32)
    invt = jnp.float32(1.0) / temperature_single[0].astype(jnp.float32)
    params = jnp.stack([thr, invt]).reshape(1, 2)

    pre, cond = pl.pallas_call(
        _entropy_body,
        grid=(8, 16),
        in_specs=[
            pl.BlockSpec(memory_space=pltpu.SMEM),
            pl.BlockSpec((1, C, 8, 128), lambda b, i: (b, 0, i, 0)),
            pl.BlockSpec((RB, 1), lambda b, i: (b * 16 + i, 0)),
        ],
        out_specs=[
            pl.BlockSpec((RB, CP), lambda b, i: (b * 16 + i, 0)),
            pl.BlockSpec((RB, 1), lambda b, i: (b * 16 + i, 0)),
        ],
        out_shape=[
            jax.ShapeDtypeStruct((N, CP), jnp.float32),
            jax.ShapeDtypeStruct((N, 1), jnp.int32),
        ],
    )(params, logits, y2)

    dest2d = pl.pallas_call(
        _dest_body,
        in_specs=[pl.BlockSpec((ROWS_2D, 128), lambda: (0, 0))],
        out_specs=pl.BlockSpec((ROWS_2D, 128), lambda: (0, 0)),
        out_shape=jax.ShapeDtypeStruct((ROWS_2D, 128), jnp.int32),
    )(cond.reshape(ROWS_2D, 128))

    out_pad = _sc_scatter()(pre, dest2d.reshape(N))
    cal_logits = out_pad[:, :C]
    cal_gt = lax.bitcast_convert_type(out_pad[:, C], jnp.int32)
    return (cal_logits, cal_gt)


# speedup vs baseline: 1.9314x; 1.0394x over previous
import functools

import jax
import jax.numpy as jnp
from jax import lax
from jax.experimental import pallas as pl
from jax.experimental.pallas import tpu as pltpu
from jax.experimental.pallas import tpu_sc as plsc

N = 131072
C = 150
CP = 256
RB = 1024
ROWS_2D = N // 128

NC = 2
NS = 16
NW = NC * NS
RPW = N // NW
G = 128
CHUNKS = RPW // G


def _entropy_body(params_ref, x_ref, g_ref, pre_ref, cond_ref):
    x = jnp.concatenate(
        [jnp.transpose(x_ref[0, :, h, :]) for h in range(8)], axis=0)
    thr = params_ref[0, 0]
    invt = params_ref[0, 1]
    m = jnp.max(x, axis=1, keepdims=True)
    e = jnp.exp(x - m)
    s = jnp.sum(e, axis=1, keepdims=True)
    t = jnp.sum(e * (x - m), axis=1, keepdims=True)
    ent = jnp.log(s) - t / s
    cond = ent < thr
    row = jnp.where(cond, x * invt, jnp.float32(1.0))
    gbits = lax.bitcast_convert_type(g_ref[...], jnp.float32)
    pad = jnp.zeros((RB, CP - C - 1), jnp.float32)
    pre_ref[...] = jnp.concatenate([row, gbits, pad], axis=1)
    cond_ref[...] = cond.astype(jnp.int32)


def _dest_body(cond_ref, dest_ref):
    c = cond_ref[...].astype(jnp.float32)
    r, l = ROWS_2D, 128
    u = (lax.broadcasted_iota(jnp.int32, (l, l), 0)
         <= lax.broadcasted_iota(jnp.int32, (l, l), 1)).astype(jnp.float32)
    cs = lax.dot_general(c, u, (((1,), (0,)), ((), ())))
    rs = jnp.sum(c, axis=1, keepdims=True)
    lo = (lax.broadcasted_iota(jnp.int32, (r, r), 0)
          > lax.broadcasted_iota(jnp.int32, (r, r), 1)).astype(jnp.float32)
    ro = lax.dot_general(lo, rs, (((1,), (0,)), ((), ())))
    cc = cs + ro
    k = jnp.max(cc)
    v = (lax.broadcasted_iota(jnp.int32, (r, l), 0) * l
         + lax.broadcasted_iota(jnp.int32, (r, l), 1)).astype(jnp.float32)
    dest = jnp.where(c > 0.5, cc - 1.0, k + v - cc)
    dest_ref[...] = dest.astype(jnp.int32)


NB = 2


def _sc_scatter_body(pre_hbm, dest_hbm, out_hbm, rows_v, idx_v, sems):
    wid = lax.axis_index("s") * NC + lax.axis_index("c")
    base0 = wid * RPW

    def issue(i, b):
        base = base0 + i * G
        pltpu.sync_copy(dest_hbm.at[pl.ds(base, G)], idx_v[b])
        pltpu.sync_copy(pre_hbm.at[pl.ds(base, G)], rows_v[b])
        pltpu.async_copy(rows_v[b], out_hbm.at[idx_v[b]], sems[b])

    for b in range(NB):
        issue(b, b)

    def round_(o, carry):
        for b in range(NB):
            i = o * NB + b
            pltpu.make_async_copy(rows_v[b], out_hbm.at[idx_v[b]],
                                  sems[b]).wait()
            issue(i, b)
        return carry

    lax.fori_loop(1, CHUNKS // NB, round_, 0)

    for b in range(NB):
        pltpu.make_async_copy(rows_v[b], out_hbm.at[idx_v[b]], sems[b]).wait()


@functools.cache
def _sc_scatter():
    return pl.kernel(
        _sc_scatter_body,
        out_type=jax.ShapeDtypeStruct((N, CP), jnp.float32),
        mesh=plsc.VectorSubcoreMesh(core_axis_name="c", subcore_axis_name="s"),
        scratch_types=[
            [pltpu.VMEM((G, CP), jnp.float32) for _ in range(NB)],
            [pltpu.VMEM((G,), jnp.int32) for _ in range(NB)],
            [pltpu.SemaphoreType.DMA for _ in range(NB)],
        ],
    )


def _extract_body(pad_ref, out_ref, gt_ref):
    out_ref[...] = pad_ref[:, :C]
    gt_ref[...] = lax.bitcast_convert_type(pad_ref[:, C:C + 1], jnp.int32)


def kernel(logits, gt, threshold, temperature_single):
    y2 = gt.reshape(N, 1)
    thr = jnp.asarray(threshold, jnp.float32)
    invt = jnp.float32(1.0) / temperature_single[0].astype(jnp.float32)
    params = jnp.stack([thr, invt]).reshape(1, 2)

    pre, cond = pl.pallas_call(
        _entropy_body,
        grid=(8, 16),
        in_specs=[
            pl.BlockSpec(memory_space=pltpu.SMEM),
            pl.BlockSpec((1, C, 8, 128), lambda b, i: (b, 0, i, 0)),
            pl.BlockSpec((RB, 1), lambda b, i: (b * 16 + i, 0)),
        ],
        out_specs=[
            pl.BlockSpec((RB, CP), lambda b, i: (b * 16 + i, 0)),
            pl.BlockSpec((RB, 1), lambda b, i: (b * 16 + i, 0)),
        ],
        out_shape=[
            jax.ShapeDtypeStruct((N, CP), jnp.float32),
            jax.ShapeDtypeStruct((N, 1), jnp.int32),
        ],
    )(params, logits, y2)

    dest2d = pl.pallas_call(
        _dest_body,
        in_specs=[pl.BlockSpec((ROWS_2D, 128), lambda: (0, 0))],
        out_specs=pl.BlockSpec((ROWS_2D, 128), lambda: (0, 0)),
        out_shape=jax.ShapeDtypeStruct((ROWS_2D, 128), jnp.int32),
    )(cond.reshape(ROWS_2D, 128))

    out_pad = _sc_scatter()(pre, dest2d.reshape(N))
    cal_logits = out_pad[:, :C]
    cal_gt = lax.bitcast_convert_type(out_pad[:, C], jnp.int32)
    return (cal_logits, cal_gt)
